# baseline (speedup 1.0000x reference)
import functools

import jax
import jax.numpy as jnp
from jax import lax
from jax.experimental import pallas as pl
from jax.experimental.pallas import tpu as pltpu
from jax.experimental.pallas import tpu_sc as plsc

EMB_DIM = 64
K = 8
B = 4096
HALF = EMB_DIM // 2
D = K * EMB_DIM

NC = 2
NS = 16
NW = NC * NS
CHUNK = 32

P = 4
E = B // P
RP = 2 * E

BLK = 1024
NBLK = E // BLK


@functools.cache
def _make_sc_gather(nex, e0):
    rows_per_w = nex // NS
    nchunks = rows_per_w // CHUNK
    nrows = 2 * nex
    mesh = plsc.VectorSubcoreMesh(
        core_axis_name="c", subcore_axis_name="s", num_cores=NC, num_subcores=NS
    )

    @functools.partial(
        pl.kernel,
        mesh=mesh,
        out_type=jax.ShapeDtypeStruct((nrows, D), jnp.float32),
        scratch_types=[
            pltpu.VMEM((rows_per_w,), jnp.int32),
            pltpu.VMEM((CHUNK, D), jnp.float32),
            pltpu.VMEM((CHUNK, D), jnp.float32),
            pltpu.VMEM((CHUNK, D), jnp.float32),
            pltpu.VMEM((CHUNK, D), jnp.float32),
            pltpu.SemaphoreType.DMA,
            pltpu.SemaphoreType.DMA,
            pltpu.SemaphoreType.DMA,
            pltpu.SemaphoreType.DMA,
            pltpu.SemaphoreType.DMA,
            pltpu.SemaphoreType.DMA,
            pltpu.SemaphoreType.DMA,
            pltpu.SemaphoreType.DMA,
        ],
    )
    def gather_k(idx_hbm, table_hbm, out_hbm, idx_v,
                 buf0, buf1, buf2, buf3, g0, g1, g2, g3, w0, w1, w2, w3):
        col = lax.axis_index("c")
        sub = lax.axis_index("s")
        src0 = col * B + e0 + sub * rows_per_w
        r0 = col * nex + sub * rows_per_w
        pltpu.sync_copy(idx_hbm.at[pl.ds(src0, rows_per_w)], idx_v)
        bufs = (buf0, buf1, buf2, buf3)
        gsem = (g0, g1, g2, g3)
        wsem = (w0, w1, w2, w3)
        nbuf = len(bufs)
        gathers = [None] * nbuf
        writes = [None] * nbuf
        for k in range(nchunks):
            bsel = k % nbuf
            if writes[bsel] is not None:
                writes[bsel].wait()
            gathers[bsel] = pltpu.async_copy(
                table_hbm.at[idx_v.at[pl.ds(k * CHUNK, CHUNK)]],
                bufs[bsel],
                gsem[bsel],
            )
            oldest = k - (nbuf - 1)
            if oldest >= 0:
                osel = oldest % nbuf
                gathers[osel].wait()
                writes[osel] = pltpu.async_copy(
                    bufs[osel],
                    out_hbm.at[pl.ds(r0 + oldest * CHUNK, CHUNK)],
                    wsem[osel],
                )
        for k in range(max(0, nchunks - (nbuf - 1)), nchunks):
            osel = k % nbuf
            gathers[osel].wait()
            writes[osel] = pltpu.async_copy(
                bufs[osel],
                out_hbm.at[pl.ds(r0 + k * CHUNK, CHUNK)],
                wsem[osel],
            )
        for w in writes:
            if w is not None:
                w.wait()

    return gather_k


def _prod_sub(x):
    n = x.shape[1]
    while n > 8:
        n //= 2
        x = x[:, :n] * x[:, n:]
    x = x * pltpu.roll(x, 4, 1)
    x = x * pltpu.roll(x, 2, 1)
    x = x * pltpu.roll(x, 1, 1)
    return x


def _block_ssq(c_ref, d_ref):
    c = c_ref[...].T.reshape(K, EMB_DIM, BLK)
    d = d_ref[...].T.reshape(K, EMB_DIM, BLK)
    cc, co = c[:, :HALF], jnp.abs(c[:, HALF:])
    dc, do = d[:, :HALF], jnp.abs(d[:, HALF:])
    c_lo, c_hi = cc - co, cc + co
    d_lo, d_hi = dc - do, dc + do

    inter8 = jnp.zeros((K, BLK), jnp.float32)
    for bi in range(K):
        lo = jnp.maximum(c_lo[bi][None], d_lo)
        hi = jnp.minimum(c_hi[bi][None], d_hi)
        side = jnp.maximum(hi - lo, 0.0)
        p = _prod_sub(side)
        inter8 = inter8 + jnp.sum(p, axis=0)
    inter_area = inter8

    ca = _prod_sub(2.0 * co)
    c_area = jnp.sum(ca, axis=0)

    loses = jnp.where(
        c_area == 0.0,
        0.0,
        jnp.where(
            jnp.isinf(c_area),
            1.0 - inter_area * 0.5,
            1.0 - inter_area / c_area,
        ),
    )
    r = jnp.maximum(loses, 0.0)
    return jnp.sum(r * r) * 0.125


def _tc_partial_body(c_ref, d_ref, o_ref):
    i = pl.program_id(0)
    partial = _block_ssq(c_ref, d_ref)

    @pl.when(i == 0)
    def _init():
        o_ref[0, 0] = 0.0

    o_ref[0, 0] += partial


def _tc_accum_body(prev_ref, c_ref, d_ref, o_ref):
    i = pl.program_id(0)
    partial = _block_ssq(c_ref, d_ref)

    @pl.when(i == 0)
    def _init():
        o_ref[0, 0] = prev_ref[0, 0]

    o_ref[0, 0] += partial


def _tc_final_body(prev_ref, c_ref, d_ref, o_ref):
    i = pl.program_id(0)
    partial = _block_ssq(c_ref, d_ref)

    @pl.when(i == 0)
    def _init():
        o_ref[0, 0] = prev_ref[0, 0]

    o_ref[0, 0] += partial

    @pl.when(i == NBLK - 1)
    def _fin():
        o_ref[0, 0] = jnp.sqrt(o_ref[0, 0])


_gather_specs = [
    pl.BlockSpec((BLK, D), lambda i: (i, 0)),
    pl.BlockSpec((BLK, D), lambda i: (i + NBLK, 0)),
]
_out_spec = pl.BlockSpec((1, 1), lambda i: (0, 0), memory_space=pltpu.SMEM)

_tc_partial = pl.pallas_call(
    _tc_partial_body,
    grid=(NBLK,),
    in_specs=_gather_specs,
    out_specs=_out_spec,
    out_shape=jax.ShapeDtypeStruct((1, 1), jnp.float32),
    compiler_params=pltpu.CompilerParams(
        dimension_semantics=("arbitrary",),
    ),
)

_tc_accum = pl.pallas_call(
    _tc_accum_body,
    grid=(NBLK,),
    in_specs=[
        pl.BlockSpec((1, 1), lambda i: (0, 0), memory_space=pltpu.SMEM)
    ] + _gather_specs,
    out_specs=_out_spec,
    out_shape=jax.ShapeDtypeStruct((1, 1), jnp.float32),
    compiler_params=pltpu.CompilerParams(
        dimension_semantics=("arbitrary",),
    ),
)

_tc_final = pl.pallas_call(
    _tc_final_body,
    grid=(NBLK,),
    in_specs=[
        pl.BlockSpec((1, 1), lambda i: (0, 0), memory_space=pltpu.SMEM)
    ] + _gather_specs,
    out_specs=_out_spec,
    out_shape=jax.ShapeDtypeStruct((1, 1), jnp.float32),
    compiler_params=pltpu.CompilerParams(
        dimension_semantics=("arbitrary",),
    ),
)


def kernel(nf1_data, class_table):
    idx = nf1_data.astype(jnp.int32)
    flat = idx.T.reshape(2 * B)
    gs = [_make_sc_gather(E, p * E)(flat, class_table) for p in range(P)]
    acc = _tc_partial(gs[0], gs[0])
    for g in gs[1:-1]:
        acc = _tc_accum(acc, g, g)
    res = _tc_final(acc, gs[-1], gs[-1])
    return res[0, 0]


# speedup vs baseline: 1.1491x; 1.1491x over previous
import functools

import jax
import jax.numpy as jnp
from jax import lax
from jax.experimental import pallas as pl
from jax.experimental.pallas import tpu as pltpu
from jax.experimental.pallas import tpu_sc as plsc

EMB_DIM = 64
K = 8
B = 4096
HALF = EMB_DIM // 2
D = K * EMB_DIM

NC = 2
NS = 16
NW = NC * NS
CHUNK = 32

P = 2
E = B // P
RP = 2 * E

BLK = 1024
NBLK = E // BLK


@functools.cache
def _make_sc_gather(nex, e0):
    rows_per_w = nex // NS
    nchunks = rows_per_w // CHUNK
    nrows = 2 * nex
    mesh = plsc.VectorSubcoreMesh(
        core_axis_name="c", subcore_axis_name="s", num_cores=NC, num_subcores=NS
    )

    @functools.partial(
        pl.kernel,
        mesh=mesh,
        out_type=jax.ShapeDtypeStruct((nrows, D), jnp.float32),
        scratch_types=[
            pltpu.VMEM((rows_per_w,), jnp.int32),
            pltpu.VMEM((CHUNK, D), jnp.float32),
            pltpu.VMEM((CHUNK, D), jnp.float32),
            pltpu.VMEM((CHUNK, D), jnp.float32),
            pltpu.VMEM((CHUNK, D), jnp.float32),
            pltpu.SemaphoreType.DMA,
            pltpu.SemaphoreType.DMA,
            pltpu.SemaphoreType.DMA,
            pltpu.SemaphoreType.DMA,
            pltpu.SemaphoreType.DMA,
            pltpu.SemaphoreType.DMA,
            pltpu.SemaphoreType.DMA,
            pltpu.SemaphoreType.DMA,
        ],
    )
    def gather_k(idx_hbm, table_hbm, out_hbm, idx_v,
                 buf0, buf1, buf2, buf3, g0, g1, g2, g3, w0, w1, w2, w3):
        col = lax.axis_index("c")
        sub = lax.axis_index("s")
        src0 = col * B + e0 + sub * rows_per_w
        r0 = col * nex + sub * rows_per_w
        pltpu.sync_copy(idx_hbm.at[pl.ds(src0, rows_per_w)], idx_v)
        bufs = (buf0, buf1, buf2, buf3)
        gsem = (g0, g1, g2, g3)
        wsem = (w0, w1, w2, w3)
        nbuf = len(bufs)
        gathers = [None] * nbuf
        writes = [None] * nbuf
        for k in range(nchunks):
            bsel = k % nbuf
            if writes[bsel] is not None:
                writes[bsel].wait()
            gathers[bsel] = pltpu.async_copy(
                table_hbm.at[idx_v.at[pl.ds(k * CHUNK, CHUNK)]],
                bufs[bsel],
                gsem[bsel],
            )
            oldest = k - (nbuf - 1)
            if oldest >= 0:
                osel = oldest % nbuf
                gathers[osel].wait()
                writes[osel] = pltpu.async_copy(
                    bufs[osel],
                    out_hbm.at[pl.ds(r0 + oldest * CHUNK, CHUNK)],
                    wsem[osel],
                )
        for k in range(max(0, nchunks - (nbuf - 1)), nchunks):
            osel = k % nbuf
            gathers[osel].wait()
            writes[osel] = pltpu.async_copy(
                bufs[osel],
                out_hbm.at[pl.ds(r0 + k * CHUNK, CHUNK)],
                wsem[osel],
            )
        for w in writes:
            if w is not None:
                w.wait()

    return gather_k


def _prod_sub(x):
    n = x.shape[1]
    while n > 8:
        n //= 2
        x = x[:, :n] * x[:, n:]
    x = x * pltpu.roll(x, 4, 1)
    x = x * pltpu.roll(x, 2, 1)
    x = x * pltpu.roll(x, 1, 1)
    return x


def _block_ssq(c_ref, d_ref):
    c = c_ref[...].T.reshape(K, EMB_DIM, BLK)
    d = d_ref[...].T.reshape(K, EMB_DIM, BLK)
    cc, co = c[:, :HALF], jnp.abs(c[:, HALF:])
    dc, do = d[:, :HALF], jnp.abs(d[:, HALF:])
    c_lo, c_hi = cc - co, cc + co
    d_lo, d_hi = dc - do, dc + do

    inter8 = jnp.zeros((K, BLK), jnp.float32)
    for bi in range(K):
        lo = jnp.maximum(c_lo[bi][None], d_lo)
        hi = jnp.minimum(c_hi[bi][None], d_hi)
        side = jnp.maximum(hi - lo, 0.0)
        p = _prod_sub(side)
        inter8 = inter8 + jnp.sum(p, axis=0)
    inter_area = inter8

    ca = _prod_sub(2.0 * co)
    c_area = jnp.sum(ca, axis=0)

    loses = jnp.where(
        c_area == 0.0,
        0.0,
        jnp.where(
            jnp.isinf(c_area),
            1.0 - inter_area * 0.5,
            1.0 - inter_area / c_area,
        ),
    )
    r = jnp.maximum(loses, 0.0)
    return jnp.sum(r * r) * 0.125


def _tc_partial_body(c_ref, d_ref, o_ref):
    i = pl.program_id(0)
    partial = _block_ssq(c_ref, d_ref)

    @pl.when(i == 0)
    def _init():
        o_ref[0, 0] = 0.0

    o_ref[0, 0] += partial


def _tc_final_body(prev_ref, c_ref, d_ref, o_ref):
    i = pl.program_id(0)
    partial = _block_ssq(c_ref, d_ref)

    @pl.when(i == 0)
    def _init():
        o_ref[0, 0] = prev_ref[0, 0]

    o_ref[0, 0] += partial

    @pl.when(i == NBLK - 1)
    def _fin():
        o_ref[0, 0] = jnp.sqrt(o_ref[0, 0])


_gather_specs = [
    pl.BlockSpec((BLK, D), lambda i: (i, 0)),
    pl.BlockSpec((BLK, D), lambda i: (i + NBLK, 0)),
]
_out_spec = pl.BlockSpec((1, 1), lambda i: (0, 0), memory_space=pltpu.SMEM)

_tc_partial = pl.pallas_call(
    _tc_partial_body,
    grid=(NBLK,),
    in_specs=_gather_specs,
    out_specs=_out_spec,
    out_shape=jax.ShapeDtypeStruct((1, 1), jnp.float32),
    compiler_params=pltpu.CompilerParams(
        dimension_semantics=("arbitrary",),
    ),
)

_tc_final = pl.pallas_call(
    _tc_final_body,
    grid=(NBLK,),
    in_specs=[
        pl.BlockSpec((1, 1), lambda i: (0, 0), memory_space=pltpu.SMEM)
    ] + _gather_specs,
    out_specs=_out_spec,
    out_shape=jax.ShapeDtypeStruct((1, 1), jnp.float32),
    compiler_params=pltpu.CompilerParams(
        dimension_semantics=("arbitrary",),
    ),
)


def kernel(nf1_data, class_table):
    idx = nf1_data.astype(jnp.int32)
    flat = idx.T.reshape(2 * B)
    g0 = _make_sc_gather(E, 0)(flat, class_table)
    g1 = _make_sc_gather(E, E)(flat, class_table)
    p0 = _tc_partial(g0, g0)
    res = _tc_final(p0, g1, g1)
    return res[0, 0]
